# baseline (speedup 1.0000x reference)
import functools
import math

import jax
import jax.numpy as jnp
from jax import lax
from jax.experimental import pallas as pl
from jax.experimental.pallas import tpu as pltpu
from jax.experimental.pallas import tpu_sc as plsc

_B, _C, _H, _W = 4, 4, 64, 64
_ROWS = _H // 2
_L = 16
_R1 = 4
_R05 = 2
_TRMAX = _ROWS + 2 * _R1

_W1 = [math.exp(-(d - _R1) ** 2 / 2.0) for d in range(2 * _R1 + 1)]
_W05 = [math.exp(-2.0 * (d - _R05) ** 2) for d in range(2 * _R05 + 1)]


def _sc_body(x_hbm, out_hbm, xv, mp, tbuf, obuf, o2buf, pub, pbuf, pbuf2,
             stat_sh, dma_sem):
    core = lax.axis_index("c")
    s = lax.axis_index("s")
    iota = lax.broadcasted_iota(jnp.int32, (_L,), 0)

    is_blur = jnp.logical_or(core == 0, s < 8)
    is_copy = jnp.logical_and(core == 1, jnp.logical_and(s >= 8, s < 12))

    g0 = (s // 8) * 8
    sg = s - g0
    b = sg // 2
    half = sg % 2
    h0 = half * _ROWS
    c_img = jnp.where(core == 1, 3, jnp.where(s < 8, 2, 1))
    oc_out = jnp.where(core == 1, 3, jnp.where(s < 8, 1, 2))

    @pl.when(is_copy)
    def _():
        bc = s - 8
        pltpu.sync_copy(x_hbm.at[bc, 0], xv)
        pltpu.sync_copy(xv, out_hbm.at[bc, 0])

    def blur_and_stats(radius, wts):
        nt = 2 * radius + 1
        tr = _ROWS + 2 * radius
        pw = _W + 2 * radius
        cp = pltpu.async_copy(x_hbm.at[b, c_img], xv, dma_sem)

        def zero_body(i, _):
            mp[pl.ds(i * _L, _L)] = jnp.zeros((_L,), jnp.float32)
            return 0
        lax.fori_loop(0, (tr * pw + _L - 1) // _L, zero_body, 0)
        cp.wait()

        def fill_body(hl, carry):
            hg = h0 - radius + hl
            validf = jnp.where(
                jnp.logical_and(hg >= 0, hg < _H), 1.0, 0.0)
            ownf = jnp.where(
                jnp.logical_and(hg >= h0, hg < h0 + _ROWS), 1.0, 0.0)
            hgc = jnp.clip(hg, 0, _H - 1)
            new = []
            for k in range(_W // _L):
                v = xv[hgc, pl.ds(k * _L, _L)]
                m = jnp.where(v > 0, 1.0, 0.0) * validf
                plsc.store_scatter(
                    mp, [hl * pw + radius + k * _L + iota], m)
                new.append(jnp.maximum(carry[k], v * ownf + (ownf - 1.0) * 1e30))
            return tuple(new)
        xmax4 = lax.fori_loop(
            0, tr, fill_body,
            tuple(jnp.full((_L,), -1e30, jnp.float32) for _ in range(4)))
        xmaxp = jnp.maximum(jnp.maximum(xmax4[0], xmax4[1]),
                            jnp.maximum(xmax4[2], xmax4[3]))

        def hpass_body(hl, _):
            base = hl * pw
            for k in range(_W // _L):
                acc = jnp.zeros((_L,), jnp.float32)
                for d in range(nt):
                    g = plsc.load_gather(mp, [base + k * _L + d + iota])
                    acc = acc + wts[d] * g
                tbuf[hl, pl.ds(k * _L, _L)] = acc
            return 0
        lax.fori_loop(0, tr, hpass_body, 0)

        def vpass_body(h, omax):
            for k in range(_W // _L):
                acc = jnp.zeros((_L,), jnp.float32)
                for d in range(nt):
                    acc = acc + wts[d] * tbuf[h + d, pl.ds(k * _L, _L)]
                obuf[h, pl.ds(k * _L, _L)] = acc
                omax = jnp.maximum(omax, acc)
            return omax
        omaxp = lax.fori_loop(0, _ROWS, vpass_body,
                              jnp.zeros((_L,), jnp.float32))

        pub[pl.ds(0, _L)] = xmaxp
        pub[pl.ds(_L, _L)] = omaxp
        pltpu.sync_copy(pub, stat_sh.at[pl.ds(s * (2 * _L), 2 * _L)])

    @pl.when(core == 0)
    def _():
        blur_and_stats(_R1, _W1)
        @pl.when(s >= 8)
        def _():
            pltpu.sync_copy(obuf, out_hbm.at[b, 4, pl.ds(h0, _ROWS)])

        @pl.when(s < 8)
        def _():
            pltpu.sync_copy(x_hbm.at[b, 1, pl.ds(h0, _ROWS)], o2buf)

    @pl.when(jnp.logical_and(core == 1, s < 8))
    def _():
        blur_and_stats(_R05, _W05)

    plsc.subcore_barrier()

    @pl.when(is_blur)
    def _():
        pltpu.sync_copy(stat_sh.at[pl.ds(g0 * (2 * _L), 16 * _L)], pbuf)

        @pl.when(jnp.logical_and(core == 0, s < 8))
        def _():
            pltpu.sync_copy(stat_sh.at[pl.ds(16 * _L, 16 * _L)], pbuf2)

        xm = pbuf[pl.ds(0, _L)]
        for i in range(1, 8):
            xm = jnp.maximum(xm, pbuf[pl.ds(i * 2 * _L, _L)])
        has_pos = lax.reduce_max(xm, axes=(0,)) > 0
        hposf = jnp.where(has_pos, 1.0, 0.0)
        om = jnp.full((_L,), -1e30, jnp.float32)
        for i in range(8):
            self_b = jnp.where(b == i // 2, 1.0, 0.0)
            om = jnp.maximum(om, pbuf[pl.ds(i * 2 * _L + _L, _L)] * self_b +
                             (self_b - 1.0) * 1e30)
        maxv = lax.reduce_max(om, axes=(0,))
        mden = jnp.where(maxv == 0, 1.0, maxv)
        ones = jnp.ones((_L,), jnp.float32)
        inv = (ones * hposf) / (ones * mden)

        def norm_body(h, _):
            for k in range(_W // _L):
                o = obuf[h, pl.ds(k * _L, _L)]
                raw = xv[h0 + h, pl.ds(k * _L, _L)]
                obuf[h, pl.ds(k * _L, _L)] = o * inv + raw * (1.0 - hposf)
            return 0
        lax.fori_loop(0, _ROWS, norm_body, 0)
        pltpu.sync_copy(obuf, out_hbm.at[b, oc_out, pl.ds(h0, _ROWS)])

        @pl.when(jnp.logical_and(core == 0, s < 8))
        def _():
            pltpu.sync_copy(out_hbm.at[b, 4, pl.ds(h0, _ROWS)],
                            tbuf.at[pl.ds(0, _ROWS)])
            xm2 = pbuf2[pl.ds(0, _L)]
            for i in range(1, 8):
                xm2 = jnp.maximum(xm2, pbuf2[pl.ds(i * 2 * _L, _L)])
            hposf2 = jnp.where(lax.reduce_max(xm2, axes=(0,)) > 0, 1.0, 0.0)
            om2 = jnp.full((_L,), -1e30, jnp.float32)
            for i in range(8):
                self_b = jnp.where(b == i // 2, 1.0, 0.0)
                om2 = jnp.maximum(
                    om2, pbuf2[pl.ds(i * 2 * _L + _L, _L)] * self_b +
                    (self_b - 1.0) * 1e30)
            maxv2 = lax.reduce_max(om2, axes=(0,))
            mden2 = jnp.where(maxv2 == 0, 1.0, maxv2)
            inv2 = (ones * hposf2) / (ones * mden2)

            def prod_body(h, _):
                for k in range(_W // _L):
                    o2n = (tbuf[h, pl.ds(k * _L, _L)] * inv2 +
                           o2buf[h, pl.ds(k * _L, _L)] * (1.0 - hposf2))
                    o2buf[h, pl.ds(k * _L, _L)] = (
                        obuf[h, pl.ds(k * _L, _L)] * o2n)
                return 0
            lax.fori_loop(0, _ROWS, prod_body, 0)
            pltpu.sync_copy(o2buf, out_hbm.at[b, 4, pl.ds(h0, _ROWS)])


@jax.jit
def kernel(x):
    mesh = plsc.VectorSubcoreMesh(
        core_axis_name="c", subcore_axis_name="s", num_cores=2,
        num_subcores=16)
    f = functools.partial(
        pl.kernel,
        mesh=mesh,
        compiler_params=pltpu.CompilerParams(needs_layout_passes=False),
        out_type=jax.ShapeDtypeStruct((_B, 5, _H, _W), jnp.float32),
        scratch_types=[
            pltpu.VMEM((_H, _W), jnp.float32),
            pltpu.VMEM((_TRMAX * (_W + 2 * _R1),), jnp.float32),
            pltpu.VMEM((_TRMAX, _W), jnp.float32),
            pltpu.VMEM((_ROWS, _W), jnp.float32),
            pltpu.VMEM((_ROWS, _W), jnp.float32),
            pltpu.VMEM((2 * _L,), jnp.float32),
            pltpu.VMEM((16 * _L,), jnp.float32),
            pltpu.VMEM((16 * _L,), jnp.float32),
            pltpu.VMEM_SHARED((32 * _L,), jnp.float32),
            pltpu.SemaphoreType.DMA,
        ],
    )(_sc_body)
    return f(x)
